# baseline (speedup 1.0000x reference)
import jax
import jax.numpy as jnp
from jax import lax
from jax.experimental import pallas as pl
from jax.experimental.pallas import tpu as pltpu
from jax.experimental.pallas import tpu_sc as plsc

N = 10000
NP = 10240
D = 128
E = 320000

NC = 2
NS = 16
NW = NC * NS
CH = 128
K = 1
HCPT = 80
NPH = 1
CPT = HCPT * NPH
EP = NW * CPT * CH
RPT = NP // NS

_f32 = jnp.float32
_i32 = jnp.int32


def _spmm_body(h_hbm, src_hbm, dst_hbm, w_hbm, out_hbm,
               src_v, dst_v, w_v, rows_v, gs, ss, acc):
    c = lax.axis_index("c")
    s = lax.axis_index("s")
    wid = s * NC + c

    zero16 = jnp.zeros((16,), _f32)

    def _zero_buf(i, carry):
        r = i // 8
        k = i % 8
        rows_v[r, pl.ds(k * 16, 16)] = zero16
        return carry

    lax.fori_loop(0, CH * 8, _zero_buf, 0)
    row0 = s * RPT
    for k in range(RPT // CH):
        pltpu.sync_copy(rows_v.at[pl.ds(0, CH)],
                        acc.at[pl.ds(row0 + k * CH, CH)])
    plsc.subcore_barrier()

    def _scale(j, off):
        def _edge16(b, carry2):
            wv16 = w_v[j, pl.ds(b * 16, 16)]
            for e2 in range(16):
                e = b * 16 + e2
                wspl = lax.gather(
                    wv16, jnp.full((16, 1), e2, _i32),
                    dimension_numbers=lax.GatherDimensionNumbers(
                        offset_dims=(), collapsed_slice_dims=(0,),
                        start_index_map=(0,)),
                    slice_sizes=(1,),
                    mode=lax.GatherScatterMode.PROMISE_IN_BOUNDS)
                for k in range(D // 16):
                    sl = pl.ds(k * 16, 16)
                    rows_v[off + e, sl] = rows_v[off + e, sl] * wspl
            return carry2

        lax.fori_loop(0, CH // 16, _edge16, 0)

    for half in range(NPH):
        h0 = (wid * NPH + half) * HCPT
        pltpu.sync_copy(src_hbm.at[pl.ds(h0, HCPT)], src_v)
        pltpu.sync_copy(dst_hbm.at[pl.ds(h0, HCPT)], dst_v)
        pltpu.sync_copy(w_hbm.at[pl.ds(h0, HCPT)], w_v)

        def _step(g, carry):
            j = g * K
            descs = [
                pltpu.async_copy(h_hbm.at[src_v.at[j + kk]],
                                 rows_v.at[pl.ds(kk * CH, CH)], gs)
                for kk in range(K)
            ]
            for d in descs:
                d.wait()
            for kk in range(K):
                _scale(j + kk, kk * CH)
            for kk in range(K):
                pltpu.sync_copy(rows_v.at[pl.ds(kk * CH, CH)],
                                acc.at[dst_v.at[j + kk]], add=True)
            return carry

        lax.fori_loop(0, HCPT // K, _step, 0)

    plsc.subcore_barrier()

    pltpu.sync_copy(acc.at[pl.ds(row0, RPT)],
                    out_hbm.at[c, pl.ds(row0, RPT)])


_spmm = pl.kernel(
    _spmm_body,
    out_type=jax.ShapeDtypeStruct((NC, NP, D), _f32),
    mesh=plsc.VectorSubcoreMesh(core_axis_name="c", subcore_axis_name="s"),
    scratch_types=[
        pltpu.VMEM((HCPT, CH), _i32),
        pltpu.VMEM((HCPT, CH), _i32),
        pltpu.VMEM((HCPT, CH), _f32),
        pltpu.VMEM((K * CH, D), _f32),
        pltpu.SemaphoreType.DMA,
        pltpu.SemaphoreType.DMA,
        pltpu.VMEM_SHARED((NP, D), _f32),
    ],
)


_RB = 1024


def _mm_body(x_ref, w_ref, o_ref):
    o_ref[...] = jnp.dot(x_ref[...], w_ref[...],
                         preferred_element_type=_f32)


def _mm2_body(p_ref, w_ref, o_ref):
    h = jnp.tanh(p_ref[0] + p_ref[1])
    o_ref[...] = jnp.dot(h, w_ref[...], preferred_element_type=_f32)


def _norm_body(p_ref, o_ref):
    t = jnp.tanh(p_ref[0] + p_ref[1])
    sq = jnp.sum(t * t, axis=1, keepdims=True)
    o_ref[...] = t * lax.rsqrt(jnp.maximum(sq, 1e-12))


_mm = pl.pallas_call(
    _mm_body,
    grid=(NP // _RB,),
    in_specs=[pl.BlockSpec((_RB, D), lambda i: (i, 0)),
              pl.BlockSpec((D, D), lambda i: (0, 0))],
    out_specs=pl.BlockSpec((_RB, D), lambda i: (i, 0)),
    out_shape=jax.ShapeDtypeStruct((NP, D), _f32),
)

_mm2 = pl.pallas_call(
    _mm2_body,
    grid=(NP // _RB,),
    in_specs=[pl.BlockSpec((NC, _RB, D), lambda i: (0, i, 0)),
              pl.BlockSpec((D, D), lambda i: (0, 0))],
    out_specs=pl.BlockSpec((_RB, D), lambda i: (i, 0)),
    out_shape=jax.ShapeDtypeStruct((NP, D), _f32),
)

_norm = pl.pallas_call(
    _norm_body,
    grid=(NP // _RB,),
    in_specs=[pl.BlockSpec((NC, _RB, D), lambda i: (0, i, 0))],
    out_specs=pl.BlockSpec((_RB, D), lambda i: (i, 0)),
    out_shape=jax.ShapeDtypeStruct((NP, D), _f32),
)


def kernel(input_embed, edge_index, edge_weight, W0, W1):
    pad = EP - E
    src = jnp.concatenate([edge_index[0], jnp.zeros((pad,), _i32)])
    dst = jnp.concatenate([edge_index[1], jnp.zeros((pad,), _i32)])
    w = jnp.concatenate([edge_weight, jnp.zeros((pad,), _f32)])
    src = src.reshape(NW * NPH * HCPT, CH)
    dst = dst.reshape(NW * NPH * HCPT, CH)
    w = w.reshape(NW * NPH * HCPT, CH)

    x = jnp.concatenate(
        [input_embed, jnp.zeros((NP - N, D), _f32)], axis=0)

    h0 = _mm(x, W0)
    p0 = _spmm(h0, src, dst, w)
    h1 = _mm2(p0, W1)
    p1 = _spmm(h1, src, dst, w)
    return _norm(p1)[:N]


# speedup vs baseline: 1.0011x; 1.0011x over previous
import jax
import jax.numpy as jnp
from jax import lax
from jax.experimental import pallas as pl
from jax.experimental.pallas import tpu as pltpu
from jax.experimental.pallas import tpu_sc as plsc

N = 10000
NP = 10240
D = 128
E = 320000

NC = 2
NS = 16
NW = NC * NS
CH = 128
K = 1
HCPT = 80
NPH = 1
CPT = HCPT * NPH
EP = NW * CPT * CH
RPT = NP // NS

_f32 = jnp.float32
_i32 = jnp.int32


def _spmm_body(h_hbm, src_hbm, dst_hbm, w_hbm, out_hbm,
               src_v, dst_v, w_v, rows_v, gs, ss, acc):
    c = lax.axis_index("c")
    s = lax.axis_index("s")
    wid = s * NC + c

    zero16 = jnp.zeros((16,), _f32)

    def _zero_buf(i, carry):
        r = i // 8
        k = i % 8
        rows_v[r, pl.ds(k * 16, 16)] = zero16
        return carry

    lax.fori_loop(0, CH * 8, _zero_buf, 0)
    row0 = s * RPT
    for k in range(RPT // CH):
        pltpu.sync_copy(rows_v, acc.at[pl.ds(row0 + k * CH, CH)])
    plsc.subcore_barrier()

    def _scale(j, off):
        def _edge16(b, carry2):
            wv16 = w_v[j, pl.ds(b * 16, 16)]
            for e2 in range(16):
                e = b * 16 + e2
                wspl = lax.gather(
                    wv16, jnp.full((16, 1), e2, _i32),
                    dimension_numbers=lax.GatherDimensionNumbers(
                        offset_dims=(), collapsed_slice_dims=(0,),
                        start_index_map=(0,)),
                    slice_sizes=(1,),
                    mode=lax.GatherScatterMode.PROMISE_IN_BOUNDS)
                for k in range(D // 16):
                    sl = pl.ds(k * 16, 16)
                    rows_v[off + e, sl] = rows_v[off + e, sl] * wspl
            return carry2

        lax.fori_loop(0, CH // 16, _edge16, 0)

    for half in range(NPH):
        h0 = (wid * NPH + half) * HCPT
        pltpu.sync_copy(src_hbm.at[pl.ds(h0, HCPT)], src_v)
        pltpu.sync_copy(dst_hbm.at[pl.ds(h0, HCPT)], dst_v)
        pltpu.sync_copy(w_hbm.at[pl.ds(h0, HCPT)], w_v)

        def _step(j, carry):
            pltpu.async_copy(h_hbm.at[src_v.at[j]], rows_v, gs).wait()
            _scale(j, 0)
            pltpu.sync_copy(rows_v, acc.at[dst_v.at[j]], add=True)
            return carry

        lax.fori_loop(0, HCPT, _step, 0)

    plsc.subcore_barrier()

    pltpu.sync_copy(acc.at[pl.ds(row0, RPT)],
                    out_hbm.at[c, pl.ds(row0, RPT)])


_spmm = pl.kernel(
    _spmm_body,
    out_type=jax.ShapeDtypeStruct((NC, NP, D), _f32),
    mesh=plsc.VectorSubcoreMesh(core_axis_name="c", subcore_axis_name="s"),
    scratch_types=[
        pltpu.VMEM((HCPT, CH), _i32),
        pltpu.VMEM((HCPT, CH), _i32),
        pltpu.VMEM((HCPT, CH), _f32),
        pltpu.VMEM((K * CH, D), _f32),
        pltpu.SemaphoreType.DMA,
        pltpu.SemaphoreType.DMA,
        pltpu.VMEM_SHARED((NP, D), _f32),
    ],
)


_RB = 1024


def _mm_body(x_ref, w_ref, o_ref):
    o_ref[...] = jnp.dot(x_ref[...], w_ref[...],
                         preferred_element_type=_f32)


def _mm2_body(p_ref, w_ref, o_ref):
    h = jnp.tanh(p_ref[0] + p_ref[1])
    o_ref[...] = jnp.dot(h, w_ref[...], preferred_element_type=_f32)


def _norm_body(p_ref, o_ref):
    t = jnp.tanh(p_ref[0] + p_ref[1])
    sq = jnp.sum(t * t, axis=1, keepdims=True)
    o_ref[...] = t * lax.rsqrt(jnp.maximum(sq, 1e-12))


_mm = pl.pallas_call(
    _mm_body,
    grid=(NP // _RB,),
    in_specs=[pl.BlockSpec((_RB, D), lambda i: (i, 0)),
              pl.BlockSpec((D, D), lambda i: (0, 0))],
    out_specs=pl.BlockSpec((_RB, D), lambda i: (i, 0)),
    out_shape=jax.ShapeDtypeStruct((NP, D), _f32),
)

_mm2 = pl.pallas_call(
    _mm2_body,
    grid=(NP // _RB,),
    in_specs=[pl.BlockSpec((NC, _RB, D), lambda i: (0, i, 0)),
              pl.BlockSpec((D, D), lambda i: (0, 0))],
    out_specs=pl.BlockSpec((_RB, D), lambda i: (i, 0)),
    out_shape=jax.ShapeDtypeStruct((NP, D), _f32),
)

_norm = pl.pallas_call(
    _norm_body,
    grid=(NP // _RB,),
    in_specs=[pl.BlockSpec((NC, _RB, D), lambda i: (0, i, 0))],
    out_specs=pl.BlockSpec((_RB, D), lambda i: (i, 0)),
    out_shape=jax.ShapeDtypeStruct((NP, D), _f32),
)


def kernel(input_embed, edge_index, edge_weight, W0, W1):
    pad = EP - E
    src = jnp.concatenate([edge_index[0], jnp.zeros((pad,), _i32)])
    dst = jnp.concatenate([edge_index[1], jnp.zeros((pad,), _i32)])
    w = jnp.concatenate([edge_weight, jnp.zeros((pad,), _f32)])
    src = src.reshape(NW * NPH * HCPT, CH)
    dst = dst.reshape(NW * NPH * HCPT, CH)
    w = w.reshape(NW * NPH * HCPT, CH)

    x = jnp.concatenate(
        [input_embed, jnp.zeros((NP - N, D), _f32)], axis=0)

    h0 = _mm(x, W0)
    p0 = _spmm(h0, src, dst, w)
    h1 = _mm2(p0, W1)
    p1 = _spmm(h1, src, dst, w)
    return _norm(p1)[:N]


# speedup vs baseline: 1.5246x; 1.5230x over previous
import jax
import jax.numpy as jnp
from jax import lax
from jax.experimental import pallas as pl
from jax.experimental.pallas import tpu as pltpu
from jax.experimental.pallas import tpu_sc as plsc

N = 10000
NP = 10240
D = 128
E = 320000

NC = 2
NS = 16
NW = NC * NS
CH = 128
CPT = 79
EP = NW * CPT * CH
RPT = NP // NS

_f32 = jnp.float32
_i32 = jnp.int32


def _spmm_body(h_hbm, src_hbm, dst_hbm, w_hbm, out_hbm,
               src_v, dst_v, w_v, rows_v, sem, acc):
    c = lax.axis_index("c")
    s = lax.axis_index("s")
    wid = s * NC + c

    pltpu.sync_copy(src_hbm.at[wid], src_v)
    pltpu.sync_copy(dst_hbm.at[wid], dst_v)
    pltpu.sync_copy(w_hbm.at[wid], w_v)

    zero16 = jnp.zeros((16,), _f32)

    def _zero_buf(i, carry):
        r = i // 8
        k = i % 8
        rows_v[r, pl.ds(k * 16, 16)] = zero16
        return carry

    lax.fori_loop(0, CH * 8, _zero_buf, 0)
    row0 = s * RPT
    for k in range(RPT // CH):
        pltpu.sync_copy(rows_v, acc.at[pl.ds(row0 + k * CH, CH)])
    plsc.subcore_barrier()

    def _chunk(j, carry):
        pltpu.async_copy(h_hbm.at[src_v.at[j]], rows_v, sem).wait()

        def _edge16(b, carry2):
            wv16 = w_v[j, pl.ds(b * 16, 16)]
            for e2 in range(16):
                e = b * 16 + e2
                wspl = lax.gather(
                    wv16, jnp.full((16, 1), e2, _i32),
                    dimension_numbers=lax.GatherDimensionNumbers(
                        offset_dims=(), collapsed_slice_dims=(0,),
                        start_index_map=(0,)),
                    slice_sizes=(1,),
                    mode=lax.GatherScatterMode.PROMISE_IN_BOUNDS)
                for k in range(D // 16):
                    sl = pl.ds(k * 16, 16)
                    rows_v[e, sl] = rows_v[e, sl] * wspl
            return carry2

        lax.fori_loop(0, CH // 16, _edge16, 0)
        pltpu.sync_copy(rows_v, acc.at[dst_v.at[j]], add=True)
        return carry

    lax.fori_loop(0, CPT, _chunk, 0)
    plsc.subcore_barrier()

    pltpu.sync_copy(acc.at[pl.ds(row0, RPT)],
                    out_hbm.at[c, pl.ds(row0, RPT)])


_spmm = pl.kernel(
    _spmm_body,
    out_type=jax.ShapeDtypeStruct((NC, NP, D), _f32),
    mesh=plsc.VectorSubcoreMesh(core_axis_name="c", subcore_axis_name="s"),
    scratch_types=[
        pltpu.VMEM((CPT, CH), _i32),
        pltpu.VMEM((CPT, CH), _i32),
        pltpu.VMEM((CPT, CH), _f32),
        pltpu.VMEM((CH, D), _f32),
        pltpu.SemaphoreType.DMA,
        pltpu.VMEM_SHARED((NP, D), _f32),
    ],
)


_RB = 1024


def _mm_body(x_ref, w_ref, o_ref):
    o_ref[...] = jnp.dot(x_ref[...], w_ref[...],
                         preferred_element_type=_f32)


def _mm2_body(p_ref, w_ref, o_ref):
    h = jnp.tanh(p_ref[0] + p_ref[1])
    o_ref[...] = jnp.dot(h, w_ref[...], preferred_element_type=_f32)


def _norm_body(p_ref, o_ref):
    t = jnp.tanh(p_ref[0] + p_ref[1])
    sq = jnp.sum(t * t, axis=1, keepdims=True)
    o_ref[...] = t * lax.rsqrt(jnp.maximum(sq, 1e-12))


_mm = pl.pallas_call(
    _mm_body,
    grid=(NP // _RB,),
    in_specs=[pl.BlockSpec((_RB, D), lambda i: (i, 0)),
              pl.BlockSpec((D, D), lambda i: (0, 0))],
    out_specs=pl.BlockSpec((_RB, D), lambda i: (i, 0)),
    out_shape=jax.ShapeDtypeStruct((NP, D), _f32),
)

_mm2 = pl.pallas_call(
    _mm2_body,
    grid=(NP // _RB,),
    in_specs=[pl.BlockSpec((NC, _RB, D), lambda i: (0, i, 0)),
              pl.BlockSpec((D, D), lambda i: (0, 0))],
    out_specs=pl.BlockSpec((_RB, D), lambda i: (i, 0)),
    out_shape=jax.ShapeDtypeStruct((NP, D), _f32),
)

_norm = pl.pallas_call(
    _norm_body,
    grid=(NP // _RB,),
    in_specs=[pl.BlockSpec((NC, _RB, D), lambda i: (0, i, 0))],
    out_specs=pl.BlockSpec((_RB, D), lambda i: (i, 0)),
    out_shape=jax.ShapeDtypeStruct((NP, D), _f32),
)


def kernel(input_embed, edge_index, edge_weight, W0, W1):
    pad = EP - E
    src = jnp.concatenate([edge_index[0], jnp.zeros((pad,), _i32)])
    dst = jnp.concatenate([edge_index[1], jnp.zeros((pad,), _i32)])
    w = jnp.concatenate([edge_weight, jnp.zeros((pad,), _f32)])
    src = src.reshape(NW, CPT, CH)
    dst = dst.reshape(NW, CPT, CH)
    w = w.reshape(NW, CPT, CH)

    x = jnp.concatenate(
        [input_embed, jnp.zeros((NP - N, D), _f32)], axis=0)

    h0 = _mm(x, W0)
    p0 = _spmm(h0, src, dst, w)
    h1 = _mm2(p0, W1)
    p1 = _spmm(h1, src, dst, w)
    return _norm(p1)[:N]
